# baseline (speedup 1.0000x reference)
import functools

import jax
import jax.numpy as jnp
from jax import lax
from jax.experimental import pallas as pl
from jax.experimental.pallas import tpu as pltpu
from jax.experimental.pallas import tpu_sc as plsc

N = 10000
NPAD = 10240
E = 320000
F_IN = 128
H1 = 128
H2 = 64
C = 10

NC = 2
NS = 16
NW = NC * NS
EPB = 128
NB = 80
LDEPTH = 5
NBP = NB + LDEPTH
EPW = EPB * NB
EPAD = EPB * NBP * NW
RPT = NPAD // NS

_ZERO16 = None


def _mesh():
    return plsc.VectorSubcoreMesh(
        core_axis_name="c", subcore_axis_name="s",
        num_cores=NC, num_subcores=NS)


def _deg_sc(edges):
    def body(edges_hbm, out_hbm, sidx, didx, ones, zbuf, stage, degs, degd):
        c = lax.axis_index("c")
        s = lax.axis_index("s")
        wid = s * NC + c
        one16 = jnp.ones((16,), jnp.float32)
        zero16 = jnp.zeros((16,), jnp.float32)
        for i in range(EPB // 16):
            ones[pl.ds(i * 16, 16)] = one16
        for i in range(RPT // 16):
            zbuf[pl.ds(i * 16, 16)] = zero16
        base = s * RPT
        pltpu.sync_copy(zbuf, degs.at[pl.ds(base, RPT)])
        pltpu.sync_copy(zbuf, degd.at[pl.ds(base, RPT)])
        plsc.subcore_barrier()
        pltpu.sync_copy(edges_hbm.at[0, wid], sidx)
        pltpu.sync_copy(edges_hbm.at[1, wid], didx)

        def step(j, carry):
            pltpu.sync_copy(ones, degs.at[sidx.at[j]], add=True)
            pltpu.sync_copy(ones, degd.at[didx.at[j]], add=True)
            return carry

        lax.fori_loop(0, NBP, step, 0)
        plsc.subcore_barrier()
        pltpu.sync_copy(degs.at[pl.ds(base, RPT)], stage)
        pltpu.sync_copy(stage, out_hbm.at[c, 0, pl.ds(base, RPT)])
        pltpu.sync_copy(degd.at[pl.ds(base, RPT)], stage)
        pltpu.sync_copy(stage, out_hbm.at[c, 1, pl.ds(base, RPT)])

    fn = pl.kernel(
        body,
        out_type=jax.ShapeDtypeStruct((NC, 2, NPAD), jnp.float32),
        mesh=_mesh(),
        compiler_params=pltpu.CompilerParams(use_tc_tiling_on_sc=False),
        scratch_types=[
            pltpu.VMEM((NBP, EPB), jnp.int32),
            pltpu.VMEM((NBP, EPB), jnp.int32),
            pltpu.VMEM((EPB,), jnp.float32),
            pltpu.VMEM((RPT,), jnp.float32),
            pltpu.VMEM((RPT,), jnp.float32),
            pltpu.VMEM_SHARED((NPAD,), jnp.float32),
            pltpu.VMEM_SHARED((NPAD,), jnp.float32),
        ],
    )
    return fn(edges)


def _agg_sc(y, edges, feat):
    def body(y_hbm, edges_hbm, out_hbm, sidx, didx, rows, zbuf, agg,
             ssem, *gsems):
        c = lax.axis_index("c")
        s = lax.axis_index("s")
        wid = s * NC + c
        zero16 = jnp.zeros((16,), jnp.float32)
        for i in range(16):
            for j in range(feat // 16):
                zbuf[i, pl.ds(j * 16, 16)] = zero16
        base = s * RPT

        def zstep(k, carry):
            pltpu.sync_copy(zbuf, agg.at[pl.ds(base + k * 16, 16)])
            return carry

        lax.fori_loop(0, RPT // 16, zstep, 0)
        plsc.subcore_barrier()

        pltpu.sync_copy(edges_hbm.at[0, wid], sidx)
        pltpu.sync_copy(edges_hbm.at[1, wid], didx)

        for b in range(LDEPTH):
            pltpu.async_copy(y_hbm.at[sidx.at[b]], rows.at[b], gsems[b])

        def step(g, carry):
            for b in range(LDEPTH):
                j = g * LDEPTH + b
                pltpu.make_async_copy(
                    y_hbm.at[sidx.at[j]], rows.at[b], gsems[b]).wait()
                pltpu.async_copy(rows.at[b], agg.at[didx.at[j]], ssem,
                                 add=True)
                pltpu.make_async_copy(
                    rows.at[b], agg.at[didx.at[j]], ssem).wait()
                pltpu.async_copy(
                    y_hbm.at[sidx.at[j + LDEPTH]], rows.at[b], gsems[b])
            return carry

        lax.fori_loop(0, NB // LDEPTH, step, 0)
        for b in range(LDEPTH):
            pltpu.make_async_copy(
                y_hbm.at[sidx.at[b]], rows.at[b], gsems[b]).wait()
        plsc.subcore_barrier()

        def ostep(k, carry):
            pltpu.sync_copy(agg.at[pl.ds(base + k * EPB, EPB)], rows.at[0])
            pltpu.sync_copy(rows.at[0], out_hbm.at[c, pl.ds(base + k * EPB, EPB)])
            return carry

        lax.fori_loop(0, RPT // EPB, ostep, 0)

    fn = pl.kernel(
        body,
        out_type=jax.ShapeDtypeStruct((NC, NPAD, feat), jnp.float32),
        mesh=_mesh(),
        compiler_params=pltpu.CompilerParams(use_tc_tiling_on_sc=False),
        scratch_types=[
            pltpu.VMEM((NBP, EPB), jnp.int32),
            pltpu.VMEM((NBP, EPB), jnp.int32),
            pltpu.VMEM((LDEPTH, EPB, feat), jnp.float32),
            pltpu.VMEM((16, feat), jnp.float32),
            pltpu.VMEM_SHARED((NPAD, feat), jnp.float32),
            pltpu.SemaphoreType.DMA,
        ] + [pltpu.SemaphoreType.DMA] * LDEPTH,
    )
    return fn(y, edges)


def _invs_tc(degp):
    def body(d_ref, o_ref):
        deg = d_ref[0] + d_ref[1]
        o_ref[...] = lax.rsqrt(jnp.maximum(deg, 1.0))

    return pl.pallas_call(
        body,
        out_shape=jax.ShapeDtypeStruct((2, NPAD), jnp.float32),
    )(degp)


def _xw_scale_tc(xp, w, io_col):
    K = xp.shape[1]
    M = w.shape[1]
    MH = M // 2
    B = 1024

    def body(x_ref, w_ref, io_ref, o1_ref, o2_ref):
        xw = jnp.dot(x_ref[...], w_ref[...], preferred_element_type=jnp.float32)
        y = xw * io_ref[...]
        o1_ref[...] = y[:, :MH]
        o2_ref[...] = y[:, MH:]

    return pl.pallas_call(
        body,
        grid=(NPAD // B,),
        in_specs=[
            pl.BlockSpec((B, K), lambda i: (i, 0)),
            pl.BlockSpec((K, M), lambda i: (0, 0)),
            pl.BlockSpec((B, 1), lambda i: (i, 0)),
        ],
        out_specs=[
            pl.BlockSpec((B, MH), lambda i: (i, 0)),
            pl.BlockSpec((B, MH), lambda i: (i, 0)),
        ],
        out_shape=[
            jax.ShapeDtypeStruct((NPAD, MH), jnp.float32),
            jax.ShapeDtypeStruct((NPAD, MH), jnp.float32),
        ],
    )(xp, w, io_col)


def _layer2_tc(aggp_a, aggp_b, ii_col, io_col, b1r, w2):
    B = 1024
    HH = H1 // 2

    def body(aa_ref, ab_ref, ii_ref, io_ref, b_ref, w_ref, o_ref):
        i = pl.program_id(0)
        a = jnp.concatenate(
            [aa_ref[0] + aa_ref[1], ab_ref[0] + ab_ref[1]], axis=1)
        h = jnp.maximum(a * ii_ref[...] + b_ref[...], 0.0)
        y = jnp.dot(h, w_ref[...], preferred_element_type=jnp.float32)
        y = y * io_ref[...]
        rid = i * B + lax.broadcasted_iota(jnp.int32, (B, 1), 0)
        o_ref[...] = jnp.where(rid < N, y, 0.0)

    return pl.pallas_call(
        body,
        grid=(NPAD // B,),
        in_specs=[
            pl.BlockSpec((NC, B, HH), lambda i: (0, i, 0)),
            pl.BlockSpec((NC, B, HH), lambda i: (0, i, 0)),
            pl.BlockSpec((B, 1), lambda i: (i, 0)),
            pl.BlockSpec((B, 1), lambda i: (i, 0)),
            pl.BlockSpec((1, H1), lambda i: (0, 0)),
            pl.BlockSpec((H1, H2), lambda i: (0, 0)),
        ],
        out_specs=pl.BlockSpec((B, H2), lambda i: (i, 0)),
        out_shape=jax.ShapeDtypeStruct((NPAD, H2), jnp.float32),
    )(aggp_a, aggp_b, ii_col, io_col, b1r, w2)


def _h2_tc(aggp, ii_col, b2r):
    B = 1000

    def body(a_ref, ii_ref, b_ref, o_ref):
        a = a_ref[0] + a_ref[1]
        o_ref[...] = jnp.maximum(a * ii_ref[...] + b_ref[...], 0.0)

    return pl.pallas_call(
        body,
        grid=(N // B,),
        in_specs=[
            pl.BlockSpec((NC, B, H2), lambda i: (0, i, 0)),
            pl.BlockSpec((B, 1), lambda i: (i, 0)),
            pl.BlockSpec((1, H2), lambda i: (0, 0)),
        ],
        out_specs=pl.BlockSpec((B, H2), lambda i: (i, 0)),
        out_shape=jax.ShapeDtypeStruct((N, H2), jnp.float32),
    )(aggp, ii_col, b2r)


def _readout_tc(hflat, w3, b3r):
    BK = 16000
    nblk = (N * H2) // BK

    def body(h_ref, w_ref, b_ref, o_ref):
        i = pl.program_id(0)
        part = jnp.dot(h_ref[...], w_ref[...], preferred_element_type=jnp.float32)

        @pl.when(i == 0)
        def _():
            o_ref[...] = b_ref[...] + part

        @pl.when(i > 0)
        def _():
            o_ref[...] += part

    return pl.pallas_call(
        body,
        grid=(nblk,),
        in_specs=[
            pl.BlockSpec((1, BK), lambda i: (0, i)),
            pl.BlockSpec((BK, C), lambda i: (i, 0)),
            pl.BlockSpec((1, C), lambda i: (0, 0)),
        ],
        out_specs=pl.BlockSpec((1, C), lambda i: (0, 0)),
        out_shape=jax.ShapeDtypeStruct((1, C), jnp.float32),
    )(hflat, w3, b3r)


def kernel(inputs, edge_index, W1, b1, W2, b2, W3, b3):
    pad = NW * EPW - E
    src = jnp.concatenate(
        [edge_index[0], jnp.full((pad,), N, jnp.int32)]).reshape(NW, NB, EPB)
    dst = jnp.concatenate(
        [edge_index[1], jnp.full((pad,), N, jnp.int32)]).reshape(NW, NB, EPB)
    dpad = jnp.full((NW, NBP - NB, EPB), N, jnp.int32)
    edges = jnp.stack([jnp.concatenate([src, dpad], axis=1),
                       jnp.concatenate([dst, dpad], axis=1)])

    degp = _deg_sc(edges)
    invs = _invs_tc(degp)
    io_col = invs[0][:, None]
    ii_col = invs[1][:, None]

    xp = jnp.concatenate(
        [inputs, jnp.zeros((NPAD - N, F_IN), jnp.float32)], axis=0)
    y1a, y1b = _xw_scale_tc(xp, W1, io_col)
    aggp1a = _agg_sc(y1a, edges, H1 // 2)
    aggp1b = _agg_sc(y1b, edges, H1 // 2)
    y2 = _layer2_tc(aggp1a, aggp1b, ii_col, io_col, b1[None], W2)
    aggp2 = _agg_sc(y2, edges, H2)
    h2 = _h2_tc(aggp2, ii_col, b2[None])
    out = _readout_tc(h2.reshape(1, N * H2), W3, b3[None])
    return out.reshape(C)


# speedup vs baseline: 2.5891x; 2.5891x over previous
import functools

import jax
import jax.numpy as jnp
from jax import lax
from jax.experimental import pallas as pl
from jax.experimental.pallas import tpu as pltpu
from jax.experimental.pallas import tpu_sc as plsc

N = 10000
NPAD = 10240
E = 320000
F_IN = 128
H1 = 128
H2 = 64
C = 10

NC = 2
NS = 16
NW = NC * NS
EPB = 128
NB = 80
NBP = NB
EPW = EPB * NB
RPT = NPAD // NS

_ZERO16 = None


def _mesh():
    return plsc.VectorSubcoreMesh(
        core_axis_name="c", subcore_axis_name="s",
        num_cores=NC, num_subcores=NS)


def _deg_sc(edges):
    def body(edges_hbm, out_hbm, sidx, didx, ones, zbuf, stage, degs, degd):
        c = lax.axis_index("c")
        s = lax.axis_index("s")
        wid = s * NC + c
        one16 = jnp.ones((16,), jnp.float32)
        zero16 = jnp.zeros((16,), jnp.float32)
        for i in range(EPB // 16):
            ones[pl.ds(i * 16, 16)] = one16
        for i in range(RPT // 16):
            zbuf[pl.ds(i * 16, 16)] = zero16
        base = s * RPT
        pltpu.sync_copy(zbuf, degs.at[pl.ds(base, RPT)])
        pltpu.sync_copy(zbuf, degd.at[pl.ds(base, RPT)])
        plsc.subcore_barrier()
        pltpu.sync_copy(edges_hbm.at[0, wid], sidx)
        pltpu.sync_copy(edges_hbm.at[1, wid], didx)

        def step(j, carry):
            pltpu.sync_copy(ones, degs.at[sidx.at[j]], add=True)
            pltpu.sync_copy(ones, degd.at[didx.at[j]], add=True)
            return carry

        lax.fori_loop(0, NBP, step, 0)
        plsc.subcore_barrier()
        pltpu.sync_copy(degs.at[pl.ds(base, RPT)], stage)
        pltpu.sync_copy(stage, out_hbm.at[c, 0, pl.ds(base, RPT)])
        pltpu.sync_copy(degd.at[pl.ds(base, RPT)], stage)
        pltpu.sync_copy(stage, out_hbm.at[c, 1, pl.ds(base, RPT)])

    fn = pl.kernel(
        body,
        out_type=jax.ShapeDtypeStruct((NC, 2, NPAD), jnp.float32),
        mesh=_mesh(),
        compiler_params=pltpu.CompilerParams(use_tc_tiling_on_sc=False),
        scratch_types=[
            pltpu.VMEM((NBP, EPB), jnp.int32),
            pltpu.VMEM((NBP, EPB), jnp.int32),
            pltpu.VMEM((EPB,), jnp.float32),
            pltpu.VMEM((RPT,), jnp.float32),
            pltpu.VMEM((RPT,), jnp.float32),
            pltpu.VMEM_SHARED((NPAD,), jnp.float32),
            pltpu.VMEM_SHARED((NPAD,), jnp.float32),
        ],
    )
    return fn(edges)


def _agg_sc(y, edges, feat, staged):
    def body(y_hbm, edges_hbm, out_hbm, sidx, didx, rows, zbuf, agg,
             *maybe_ytab):
        c = lax.axis_index("c")
        s = lax.axis_index("s")
        wid = s * NC + c
        zero16 = jnp.zeros((16,), jnp.float32)
        for i in range(16):
            for j in range(feat // 16):
                zbuf[i, pl.ds(j * 16, 16)] = zero16
        base = s * RPT

        def zstep(k, carry):
            pltpu.sync_copy(zbuf, agg.at[pl.ds(base + k * 16, 16)])
            return carry

        lax.fori_loop(0, RPT // 16, zstep, 0)
        if staged:
            ytab = maybe_ytab[0]

            def ystep(k, carry):
                pltpu.sync_copy(y_hbm.at[pl.ds(base + k * EPB, EPB)], rows)
                pltpu.sync_copy(rows, ytab.at[pl.ds(base + k * EPB, EPB)])
                return carry

            lax.fori_loop(0, RPT // EPB, ystep, 0)
            src_tab = ytab
        else:
            src_tab = y_hbm
        plsc.subcore_barrier()
        pltpu.sync_copy(edges_hbm.at[0, wid, pl.ds(0, NB)], sidx)
        pltpu.sync_copy(edges_hbm.at[1, wid, pl.ds(0, NB)], didx)

        def step(j, carry):
            pltpu.sync_copy(src_tab.at[sidx.at[j]], rows)
            pltpu.sync_copy(rows, agg.at[didx.at[j]], add=True)
            return carry

        lax.fori_loop(0, NB, step, 0)
        plsc.subcore_barrier()

        def ostep(k, carry):
            pltpu.sync_copy(agg.at[pl.ds(base + k * EPB, EPB)], rows)
            pltpu.sync_copy(rows, out_hbm.at[c, pl.ds(base + k * EPB, EPB)])
            return carry

        lax.fori_loop(0, RPT // EPB, ostep, 0)

    fn = pl.kernel(
        body,
        out_type=jax.ShapeDtypeStruct((NC, NPAD, feat), jnp.float32),
        mesh=_mesh(),
        compiler_params=pltpu.CompilerParams(use_tc_tiling_on_sc=False),
        scratch_types=[
            pltpu.VMEM((NB, EPB), jnp.int32),
            pltpu.VMEM((NB, EPB), jnp.int32),
            pltpu.VMEM((EPB, feat), jnp.float32),
            pltpu.VMEM((16, feat), jnp.float32),
            pltpu.VMEM_SHARED((NPAD, feat), jnp.float32),
        ] + ([pltpu.VMEM_SHARED((NPAD, feat), jnp.float32)] if staged else []),
    )
    return fn(y, edges)


def _invs_tc(degp):
    def body(d_ref, o_ref):
        deg = d_ref[0] + d_ref[1]
        o_ref[...] = lax.rsqrt(jnp.maximum(deg, 1.0))

    return pl.pallas_call(
        body,
        out_shape=jax.ShapeDtypeStruct((2, NPAD), jnp.float32),
    )(degp)


def _xw_scale_tc(xp, w, io_col):
    K = xp.shape[1]
    M = w.shape[1]
    B = 1024

    def body(x_ref, w_ref, io_ref, o_ref):
        xw = jnp.dot(x_ref[...], w_ref[...], preferred_element_type=jnp.float32)
        o_ref[...] = xw * io_ref[...]

    return pl.pallas_call(
        body,
        grid=(NPAD // B,),
        in_specs=[
            pl.BlockSpec((B, K), lambda i: (i, 0)),
            pl.BlockSpec((K, M), lambda i: (0, 0)),
            pl.BlockSpec((B, 1), lambda i: (i, 0)),
        ],
        out_specs=pl.BlockSpec((B, M), lambda i: (i, 0)),
        out_shape=jax.ShapeDtypeStruct((NPAD, M), jnp.float32),
    )(xp, w, io_col)


def _layer2_tc(aggp, ii_col, io_col, b1r, w2):
    B = 1024

    def body(a_ref, ii_ref, io_ref, b_ref, w_ref, o_ref):
        i = pl.program_id(0)
        a = a_ref[0] + a_ref[1]
        h = jnp.maximum(a * ii_ref[...] + b_ref[...], 0.0)
        y = jnp.dot(h, w_ref[...], preferred_element_type=jnp.float32)
        y = y * io_ref[...]
        rid = i * B + lax.broadcasted_iota(jnp.int32, (B, 1), 0)
        o_ref[...] = jnp.where(rid < N, y, 0.0)

    return pl.pallas_call(
        body,
        grid=(NPAD // B,),
        in_specs=[
            pl.BlockSpec((NC, B, H1), lambda i: (0, i, 0)),
            pl.BlockSpec((B, 1), lambda i: (i, 0)),
            pl.BlockSpec((B, 1), lambda i: (i, 0)),
            pl.BlockSpec((1, H1), lambda i: (0, 0)),
            pl.BlockSpec((H1, H2), lambda i: (0, 0)),
        ],
        out_specs=pl.BlockSpec((B, H2), lambda i: (i, 0)),
        out_shape=jax.ShapeDtypeStruct((NPAD, H2), jnp.float32),
    )(aggp, ii_col, io_col, b1r, w2)


def _h2_tc(aggp, ii_col, b2r):
    B = 1000

    def body(a_ref, ii_ref, b_ref, o_ref):
        a = a_ref[0] + a_ref[1]
        o_ref[...] = jnp.maximum(a * ii_ref[...] + b_ref[...], 0.0)

    return pl.pallas_call(
        body,
        grid=(N // B,),
        in_specs=[
            pl.BlockSpec((NC, B, H2), lambda i: (0, i, 0)),
            pl.BlockSpec((B, 1), lambda i: (i, 0)),
            pl.BlockSpec((1, H2), lambda i: (0, 0)),
        ],
        out_specs=pl.BlockSpec((B, H2), lambda i: (i, 0)),
        out_shape=jax.ShapeDtypeStruct((N, H2), jnp.float32),
    )(aggp, ii_col, b2r)


def _readout_tc(hflat, w3, b3r):
    BK = 16000
    nblk = (N * H2) // BK

    def body(h_ref, w_ref, b_ref, o_ref):
        i = pl.program_id(0)
        part = jnp.dot(h_ref[...], w_ref[...], preferred_element_type=jnp.float32)

        @pl.when(i == 0)
        def _():
            o_ref[...] = b_ref[...] + part

        @pl.when(i > 0)
        def _():
            o_ref[...] += part

    return pl.pallas_call(
        body,
        grid=(nblk,),
        in_specs=[
            pl.BlockSpec((1, BK), lambda i: (0, i)),
            pl.BlockSpec((BK, C), lambda i: (i, 0)),
            pl.BlockSpec((1, C), lambda i: (0, 0)),
        ],
        out_specs=pl.BlockSpec((1, C), lambda i: (0, 0)),
        out_shape=jax.ShapeDtypeStruct((1, C), jnp.float32),
    )(hflat, w3, b3r)


def kernel(inputs, edge_index, W1, b1, W2, b2, W3, b3):
    pad = NW * EPW - E
    src = jnp.concatenate(
        [edge_index[0], jnp.full((pad,), N, jnp.int32)]).reshape(NW, NB, EPB)
    dst = jnp.concatenate(
        [edge_index[1], jnp.full((pad,), N, jnp.int32)]).reshape(NW, NB, EPB)
    edges = jnp.stack([src, dst])

    degp = _deg_sc(edges)
    invs = _invs_tc(degp)
    io_col = invs[0][:, None]
    ii_col = invs[1][:, None]

    xp = jnp.concatenate(
        [inputs, jnp.zeros((NPAD - N, F_IN), jnp.float32)], axis=0)
    y1 = _xw_scale_tc(xp, W1, io_col)
    aggp1 = _agg_sc(y1, edges, H1, staged=False)
    y2 = _layer2_tc(aggp1, ii_col, io_col, b1[None], W2)
    aggp2 = _agg_sc(y2, edges, H2, staged=True)
    h2 = _h2_tc(aggp2, ii_col, b2[None])
    out = _readout_tc(h2.reshape(1, N * H2), W3, b3[None])
    return out.reshape(C)


# speedup vs baseline: 3.5476x; 1.3702x over previous
import functools

import jax
import jax.numpy as jnp
from jax import lax
from jax.experimental import pallas as pl
from jax.experimental.pallas import tpu as pltpu
from jax.experimental.pallas import tpu_sc as plsc

N = 10000
NPAD = 10240
E = 320000
F_IN = 128
H1 = 128
H2 = 64
C = 10

NC = 2
NS = 16
NW = NC * NS
EPB = 128
NB = 80
NBP = NB
EPW = EPB * NB
RPT = NPAD // NS

_ZERO16 = None


def _mesh():
    return plsc.VectorSubcoreMesh(
        core_axis_name="c", subcore_axis_name="s",
        num_cores=NC, num_subcores=NS)


def _deg_sc(edges):
    def body(edges_hbm, out_hbm, sidx, didx, ones, zbuf, stage, degs, degd):
        c = lax.axis_index("c")
        s = lax.axis_index("s")
        wid = s * NC + c
        one16 = jnp.ones((16,), jnp.float32)
        zero16 = jnp.zeros((16,), jnp.float32)
        for i in range(EPB // 16):
            ones[pl.ds(i * 16, 16)] = one16
        for i in range(RPT // 16):
            zbuf[pl.ds(i * 16, 16)] = zero16
        base = s * RPT
        pltpu.sync_copy(zbuf, degs.at[pl.ds(base, RPT)])
        pltpu.sync_copy(zbuf, degd.at[pl.ds(base, RPT)])
        plsc.subcore_barrier()
        pltpu.sync_copy(edges_hbm.at[0, wid], sidx)
        pltpu.sync_copy(edges_hbm.at[1, wid], didx)

        def step(j, carry):
            pltpu.sync_copy(ones, degs.at[sidx.at[j]], add=True)
            pltpu.sync_copy(ones, degd.at[didx.at[j]], add=True)
            return carry

        lax.fori_loop(0, NBP, step, 0)
        plsc.subcore_barrier()
        pltpu.sync_copy(degs.at[pl.ds(base, RPT)], stage)
        pltpu.sync_copy(stage, out_hbm.at[c, 0, pl.ds(base, RPT)])
        pltpu.sync_copy(degd.at[pl.ds(base, RPT)], stage)
        pltpu.sync_copy(stage, out_hbm.at[c, 1, pl.ds(base, RPT)])

    fn = pl.kernel(
        body,
        out_type=jax.ShapeDtypeStruct((NC, 2, NPAD), jnp.float32),
        mesh=_mesh(),
        compiler_params=pltpu.CompilerParams(use_tc_tiling_on_sc=False),
        scratch_types=[
            pltpu.VMEM((NBP, EPB), jnp.int32),
            pltpu.VMEM((NBP, EPB), jnp.int32),
            pltpu.VMEM((EPB,), jnp.float32),
            pltpu.VMEM((RPT,), jnp.float32),
            pltpu.VMEM((RPT,), jnp.float32),
            pltpu.VMEM_SHARED((NPAD,), jnp.float32),
            pltpu.VMEM_SHARED((NPAD,), jnp.float32),
        ],
    )
    return fn(edges)


def _agg_sc(y, edges, feat, staged):
    def body(y_hbm, edges_hbm, out_hbm, sidx, didx, rows, zbuf, agg,
             *maybe_ytab):
        c = lax.axis_index("c")
        s = lax.axis_index("s")
        wid = s * NC + c
        zero16 = jnp.zeros((16,), jnp.float32)
        for i in range(16):
            for j in range(feat // 16):
                zbuf[i, pl.ds(j * 16, 16)] = zero16
        base = s * RPT

        def zstep(k, carry):
            pltpu.sync_copy(zbuf, agg.at[pl.ds(base + k * 16, 16)])
            return carry

        lax.fori_loop(0, RPT // 16, zstep, 0)
        if staged:
            ytab = maybe_ytab[0]

            def ystep(k, carry):
                pltpu.sync_copy(y_hbm.at[pl.ds(base + k * EPB, EPB)], rows)
                pltpu.sync_copy(rows, ytab.at[pl.ds(base + k * EPB, EPB)])
                return carry

            lax.fori_loop(0, RPT // EPB, ystep, 0)
            src_tab = ytab
        else:
            src_tab = y_hbm
        plsc.subcore_barrier()
        pltpu.sync_copy(edges_hbm.at[0, wid, pl.ds(0, NB)], sidx)
        pltpu.sync_copy(edges_hbm.at[1, wid, pl.ds(0, NB)], didx)

        def step(j, carry):
            pltpu.sync_copy(src_tab.at[sidx.at[j]], rows)
            pltpu.sync_copy(rows, agg.at[didx.at[j]], add=True)
            return carry

        lax.fori_loop(0, NB, step, 0)
        plsc.subcore_barrier()

        def ostep(k, carry):
            pltpu.sync_copy(agg.at[pl.ds(base + k * EPB, EPB)], rows)
            pltpu.sync_copy(rows, out_hbm.at[c, pl.ds(base + k * EPB, EPB)])
            return carry

        lax.fori_loop(0, RPT // EPB, ostep, 0)

    fn = pl.kernel(
        body,
        out_type=jax.ShapeDtypeStruct((NC, NPAD, feat), jnp.float32),
        mesh=_mesh(),
        compiler_params=pltpu.CompilerParams(use_tc_tiling_on_sc=False),
        scratch_types=[
            pltpu.VMEM((NB, EPB), jnp.int32),
            pltpu.VMEM((NB, EPB), jnp.int32),
            pltpu.VMEM((EPB, feat), jnp.float32),
            pltpu.VMEM((16, feat), jnp.float32),
            pltpu.VMEM_SHARED((NPAD, feat), jnp.float32),
        ] + ([pltpu.VMEM_SHARED((NPAD, feat), jnp.float32)] if staged else []),
    )
    return fn(y, edges)


def _invs_tc(degp):
    def body(d_ref, o_ref):
        deg = d_ref[0] + d_ref[1]
        o_ref[...] = lax.rsqrt(jnp.maximum(deg, 1.0))

    return pl.pallas_call(
        body,
        out_shape=jax.ShapeDtypeStruct((2, NPAD), jnp.float32),
    )(degp)


def _xw_scale_tc(xp, w, io_col):
    K = xp.shape[1]
    M = w.shape[1]
    MH = M // 2
    B = 1024

    def body(x_ref, w_ref, io_ref, o1_ref, o2_ref):
        xw = jnp.dot(x_ref[...], w_ref[...], preferred_element_type=jnp.float32)
        y = xw * io_ref[...]
        o1_ref[...] = y[:, :MH]
        o2_ref[...] = y[:, MH:]

    return pl.pallas_call(
        body,
        grid=(NPAD // B,),
        in_specs=[
            pl.BlockSpec((B, K), lambda i: (i, 0)),
            pl.BlockSpec((K, M), lambda i: (0, 0)),
            pl.BlockSpec((B, 1), lambda i: (i, 0)),
        ],
        out_specs=[
            pl.BlockSpec((B, MH), lambda i: (i, 0)),
            pl.BlockSpec((B, MH), lambda i: (i, 0)),
        ],
        out_shape=[
            jax.ShapeDtypeStruct((NPAD, MH), jnp.float32),
            jax.ShapeDtypeStruct((NPAD, MH), jnp.float32),
        ],
    )(xp, w, io_col)


def _layer2_tc(aggp_a, aggp_b, ii_col, io_col, b1r, w2):
    B = 1024

    def body(aa_ref, ab_ref, ii_ref, io_ref, b_ref, w_ref, o_ref):
        i = pl.program_id(0)
        a = jnp.concatenate(
            [aa_ref[0] + aa_ref[1], ab_ref[0] + ab_ref[1]], axis=1)
        h = jnp.maximum(a * ii_ref[...] + b_ref[...], 0.0)
        y = jnp.dot(h, w_ref[...], preferred_element_type=jnp.float32)
        y = y * io_ref[...]
        rid = i * B + lax.broadcasted_iota(jnp.int32, (B, 1), 0)
        o_ref[...] = jnp.where(rid < N, y, 0.0)

    return pl.pallas_call(
        body,
        grid=(NPAD // B,),
        in_specs=[
            pl.BlockSpec((NC, B, H1 // 2), lambda i: (0, i, 0)),
            pl.BlockSpec((NC, B, H1 // 2), lambda i: (0, i, 0)),
            pl.BlockSpec((B, 1), lambda i: (i, 0)),
            pl.BlockSpec((B, 1), lambda i: (i, 0)),
            pl.BlockSpec((1, H1), lambda i: (0, 0)),
            pl.BlockSpec((H1, H2), lambda i: (0, 0)),
        ],
        out_specs=pl.BlockSpec((B, H2), lambda i: (i, 0)),
        out_shape=jax.ShapeDtypeStruct((NPAD, H2), jnp.float32),
    )(aggp_a, aggp_b, ii_col, io_col, b1r, w2)


def _h2_tc(aggp, ii_col, b2r):
    B = 1000

    def body(a_ref, ii_ref, b_ref, o_ref):
        a = a_ref[0] + a_ref[1]
        o_ref[...] = jnp.maximum(a * ii_ref[...] + b_ref[...], 0.0)

    return pl.pallas_call(
        body,
        grid=(N // B,),
        in_specs=[
            pl.BlockSpec((NC, B, H2), lambda i: (0, i, 0)),
            pl.BlockSpec((B, 1), lambda i: (i, 0)),
            pl.BlockSpec((1, H2), lambda i: (0, 0)),
        ],
        out_specs=pl.BlockSpec((B, H2), lambda i: (i, 0)),
        out_shape=jax.ShapeDtypeStruct((N, H2), jnp.float32),
    )(aggp, ii_col, b2r)


def _readout_tc(h2, w3b, b3r):
    B = 1000
    W = H2 * C

    def body(h_ref, w_ref, b_ref, o_ref):
        i = pl.program_id(0)
        m = lax.dot_general(h_ref[...], w_ref[...], (((0,), (0,)), ((), ())),
                            preferred_element_type=jnp.float32)
        k_iota = lax.broadcasted_iota(jnp.int32, (H2, W), 1)
        f_iota = lax.broadcasted_iota(jnp.int32, (H2, W), 0)
        stripe = k_iota - C * f_iota
        parts = [
            jnp.sum(jnp.where(stripe == c, m, 0.0), dtype=jnp.float32)
            for c in range(C)
        ]
        part = jnp.stack(parts).reshape(1, C)

        @pl.when(i == 0)
        def _():
            o_ref[...] = b_ref[...] + part

        @pl.when(i > 0)
        def _():
            o_ref[...] += part

    return pl.pallas_call(
        body,
        grid=(N // B,),
        in_specs=[
            pl.BlockSpec((B, H2), lambda i: (i, 0)),
            pl.BlockSpec((B, W), lambda i: (i, 0)),
            pl.BlockSpec((1, C), lambda i: (0, 0)),
        ],
        out_specs=pl.BlockSpec((1, C), lambda i: (0, 0)),
        out_shape=jax.ShapeDtypeStruct((1, C), jnp.float32),
    )(h2, w3b, b3r)


def kernel(inputs, edge_index, W1, b1, W2, b2, W3, b3):
    pad = NW * EPW - E
    src = jnp.concatenate(
        [edge_index[0], jnp.full((pad,), N, jnp.int32)]).reshape(NW, NB, EPB)
    dst = jnp.concatenate(
        [edge_index[1], jnp.full((pad,), N, jnp.int32)]).reshape(NW, NB, EPB)
    edges = jnp.stack([src, dst])

    degp = _deg_sc(edges)
    invs = _invs_tc(degp)
    io_col = invs[0][:, None]
    ii_col = invs[1][:, None]

    xp = jnp.concatenate(
        [inputs, jnp.zeros((NPAD - N, F_IN), jnp.float32)], axis=0)
    y1a, y1b = _xw_scale_tc(xp, W1, io_col)
    aggp1a = _agg_sc(y1a, edges, H1 // 2, staged=True)
    aggp1b = _agg_sc(y1b, edges, H1 // 2, staged=True)
    y2 = _layer2_tc(aggp1a, aggp1b, ii_col, io_col, b1[None], W2)
    aggp2 = _agg_sc(y2, edges, H2, staged=True)
    h2 = _h2_tc(aggp2, ii_col, b2[None])
    out = _readout_tc(h2, W3.reshape(N, H2 * C), b3[None])
    return out.reshape(C)


# speedup vs baseline: 4.2759x; 1.2053x over previous
import functools

import jax
import jax.numpy as jnp
from jax import lax
from jax.experimental import pallas as pl
from jax.experimental.pallas import tpu as pltpu
from jax.experimental.pallas import tpu_sc as plsc

N = 10000
NPAD = 10240
E = 320000
F_IN = 128
H1 = 128
H2 = 64
C = 10

NC = 2
NS = 16
NW = NC * NS
EPB = 128
NB = 80
NBP = NB
EPW = EPB * NB
RPT = NPAD // NS

_ZERO16 = None


def _mesh():
    return plsc.VectorSubcoreMesh(
        core_axis_name="c", subcore_axis_name="s",
        num_cores=NC, num_subcores=NS)


def _deg_sc(edges):
    def body(edges_hbm, out_hbm, sidx, didx, ones, zbuf, stage, degs, degd):
        c = lax.axis_index("c")
        s = lax.axis_index("s")
        wid = s * NC + c
        one16 = jnp.ones((16,), jnp.float32)
        zero16 = jnp.zeros((16,), jnp.float32)
        for i in range(EPB // 16):
            ones[pl.ds(i * 16, 16)] = one16
        for i in range(RPT // 16):
            zbuf[pl.ds(i * 16, 16)] = zero16
        base = s * RPT
        pltpu.sync_copy(zbuf, degs.at[pl.ds(base, RPT)])
        pltpu.sync_copy(zbuf, degd.at[pl.ds(base, RPT)])
        plsc.subcore_barrier()
        pltpu.sync_copy(edges_hbm.at[0, wid], sidx)
        pltpu.sync_copy(edges_hbm.at[1, wid], didx)

        def step(j, carry):
            pltpu.sync_copy(ones, degs.at[sidx.at[j]], add=True)
            pltpu.sync_copy(ones, degd.at[didx.at[j]], add=True)
            return carry

        lax.fori_loop(0, NBP, step, 0)
        plsc.subcore_barrier()
        pltpu.sync_copy(degs.at[pl.ds(base, RPT)], stage)
        pltpu.sync_copy(stage, out_hbm.at[c, 0, pl.ds(base, RPT)])
        pltpu.sync_copy(degd.at[pl.ds(base, RPT)], stage)
        pltpu.sync_copy(stage, out_hbm.at[c, 1, pl.ds(base, RPT)])

    fn = pl.kernel(
        body,
        out_type=jax.ShapeDtypeStruct((NC, 2, NPAD), jnp.float32),
        mesh=_mesh(),
        compiler_params=pltpu.CompilerParams(use_tc_tiling_on_sc=False),
        scratch_types=[
            pltpu.VMEM((NBP, EPB), jnp.int32),
            pltpu.VMEM((NBP, EPB), jnp.int32),
            pltpu.VMEM((EPB,), jnp.float32),
            pltpu.VMEM((RPT,), jnp.float32),
            pltpu.VMEM((RPT,), jnp.float32),
            pltpu.VMEM_SHARED((NPAD,), jnp.float32),
            pltpu.VMEM_SHARED((NPAD,), jnp.float32),
        ],
    )
    return fn(edges)


def _agg_sc(y, edges, feat, staged):
    def body(y_hbm, edges_hbm, out_hbm, sidx, didx, rows, zbuf, agg,
             *maybe_ytab):
        c = lax.axis_index("c")
        s = lax.axis_index("s")
        wid = s * NC + c
        zero16 = jnp.zeros((16,), jnp.float32)
        for i in range(16):
            for j in range(feat // 16):
                zbuf[i, pl.ds(j * 16, 16)] = zero16
        base = s * RPT

        def zstep(k, carry):
            pltpu.sync_copy(zbuf, agg.at[pl.ds(base + k * 16, 16)])
            return carry

        lax.fori_loop(0, RPT // 16, zstep, 0)
        if staged:
            ytab = maybe_ytab[0]

            def ystep(k, carry):
                pltpu.sync_copy(y_hbm.at[pl.ds(base + k * EPB, EPB)], rows)
                pltpu.sync_copy(rows, ytab.at[pl.ds(base + k * EPB, EPB)])
                return carry

            lax.fori_loop(0, RPT // EPB, ystep, 0)
            src_tab = ytab
        else:
            src_tab = y_hbm
        plsc.subcore_barrier()
        pltpu.sync_copy(edges_hbm.at[0, wid, pl.ds(0, NB)], sidx)
        pltpu.sync_copy(edges_hbm.at[1, wid, pl.ds(0, NB)], didx)

        def step(j, carry):
            pltpu.sync_copy(src_tab.at[sidx.at[j]], rows)
            pltpu.sync_copy(rows, agg.at[didx.at[j]], add=True)
            return carry

        lax.fori_loop(0, NB, step, 0)
        plsc.subcore_barrier()

        def ostep(k, carry):
            pltpu.sync_copy(agg.at[pl.ds(base + k * EPB, EPB)], rows)
            pltpu.sync_copy(rows, out_hbm.at[c, pl.ds(base + k * EPB, EPB)])
            return carry

        lax.fori_loop(0, RPT // EPB, ostep, 0)

    fn = pl.kernel(
        body,
        out_type=jax.ShapeDtypeStruct((NC, NPAD, feat), jnp.float32),
        mesh=_mesh(),
        compiler_params=pltpu.CompilerParams(use_tc_tiling_on_sc=False),
        scratch_types=[
            pltpu.VMEM((NB, EPB), jnp.int32),
            pltpu.VMEM((NB, EPB), jnp.int32),
            pltpu.VMEM((EPB, feat), jnp.float32),
            pltpu.VMEM((16, feat), jnp.float32),
            pltpu.VMEM_SHARED((NPAD, feat), jnp.float32),
        ] + ([pltpu.VMEM_SHARED((NPAD, feat), jnp.float32)] if staged else []),
    )
    return fn(y, edges)


def _agg2x_sc(ya, yb, edges):
    feat = H1 // 2

    def body(ya_hbm, yb_hbm, edges_hbm, outa_hbm, outb_hbm,
             sidx, didx, rows, zbuf, ytab, agg):
        c = lax.axis_index("c")
        s = lax.axis_index("s")
        wid = s * NC + c
        zero16 = jnp.zeros((16,), jnp.float32)
        for i in range(16):
            for j in range(feat // 16):
                zbuf[i, pl.ds(j * 16, 16)] = zero16
        base = s * RPT
        pltpu.sync_copy(edges_hbm.at[0, wid, pl.ds(0, NB)], sidx)
        pltpu.sync_copy(edges_hbm.at[1, wid, pl.ds(0, NB)], didx)

        for y_hbm, out_hbm in ((ya_hbm, outa_hbm), (yb_hbm, outb_hbm)):
            def zstep(k, carry):
                pltpu.sync_copy(zbuf, agg.at[pl.ds(base + k * 16, 16)])
                return carry

            lax.fori_loop(0, RPT // 16, zstep, 0)

            def ystep(k, carry):
                pltpu.sync_copy(y_hbm.at[pl.ds(base + k * EPB, EPB)], rows)
                pltpu.sync_copy(rows, ytab.at[pl.ds(base + k * EPB, EPB)])
                return carry

            lax.fori_loop(0, RPT // EPB, ystep, 0)
            plsc.subcore_barrier()

            def step(j, carry):
                pltpu.sync_copy(ytab.at[sidx.at[j]], rows)
                pltpu.sync_copy(rows, agg.at[didx.at[j]], add=True)
                return carry

            lax.fori_loop(0, NB, step, 0)
            plsc.subcore_barrier()

            def ostep(k, carry):
                pltpu.sync_copy(agg.at[pl.ds(base + k * EPB, EPB)], rows)
                pltpu.sync_copy(rows, out_hbm.at[c, pl.ds(base + k * EPB, EPB)])
                return carry

            lax.fori_loop(0, RPT // EPB, ostep, 0)

    fn = pl.kernel(
        body,
        out_type=(jax.ShapeDtypeStruct((NC, NPAD, feat), jnp.float32),
                  jax.ShapeDtypeStruct((NC, NPAD, feat), jnp.float32)),
        mesh=_mesh(),
        compiler_params=pltpu.CompilerParams(use_tc_tiling_on_sc=False),
        scratch_types=[
            pltpu.VMEM((NB, EPB), jnp.int32),
            pltpu.VMEM((NB, EPB), jnp.int32),
            pltpu.VMEM((EPB, feat), jnp.float32),
            pltpu.VMEM((16, feat), jnp.float32),
            pltpu.VMEM_SHARED((NPAD, feat), jnp.float32),
            pltpu.VMEM_SHARED((NPAD, feat), jnp.float32),
        ],
    )
    return fn(ya, yb, edges)


def _invs_tc(degp):
    def body(d_ref, o_ref):
        deg = d_ref[0] + d_ref[1]
        o_ref[...] = lax.rsqrt(jnp.maximum(deg, 1.0))

    return pl.pallas_call(
        body,
        out_shape=jax.ShapeDtypeStruct((2, NPAD), jnp.float32),
    )(degp)


def _xw_scale_tc(xp, w, io_col):
    K = xp.shape[1]
    M = w.shape[1]
    MH = M // 2
    B = 1024

    def body(x_ref, w_ref, io_ref, o1_ref, o2_ref):
        xw = jnp.dot(x_ref[...], w_ref[...], preferred_element_type=jnp.float32)
        y = xw * io_ref[...]
        o1_ref[...] = y[:, :MH]
        o2_ref[...] = y[:, MH:]

    return pl.pallas_call(
        body,
        grid=(NPAD // B,),
        in_specs=[
            pl.BlockSpec((B, K), lambda i: (i, 0)),
            pl.BlockSpec((K, M), lambda i: (0, 0)),
            pl.BlockSpec((B, 1), lambda i: (i, 0)),
        ],
        out_specs=[
            pl.BlockSpec((B, MH), lambda i: (i, 0)),
            pl.BlockSpec((B, MH), lambda i: (i, 0)),
        ],
        out_shape=[
            jax.ShapeDtypeStruct((NPAD, MH), jnp.float32),
            jax.ShapeDtypeStruct((NPAD, MH), jnp.float32),
        ],
    )(xp, w, io_col)


def _layer2_tc(aggp_a, aggp_b, ii_col, io_col, b1r, w2, dep):
    B = 1024

    def body(aa_ref, ab_ref, ii_ref, io_ref, b_ref, w_ref, dep_ref, o_ref):
        i = pl.program_id(0)
        ii = ii_ref[...]
        ha = jnp.maximum(
            (aa_ref[0] + aa_ref[1]) * ii + b_ref[:, :H1 // 2], 0.0)
        hb = jnp.maximum(
            (ab_ref[0] + ab_ref[1]) * ii + b_ref[:, H1 // 2:], 0.0)
        y = (jnp.dot(ha, w_ref[:H1 // 2], preferred_element_type=jnp.float32)
             + jnp.dot(hb, w_ref[H1 // 2:], preferred_element_type=jnp.float32))
        y = y * io_ref[...]
        rid = i * B + lax.broadcasted_iota(jnp.int32, (B, 1), 0)
        o_ref[...] = jnp.where(rid < N, y, 0.0)

    return pl.pallas_call(
        body,
        grid=(NPAD // B,),
        in_specs=[
            pl.BlockSpec((NC, B, H1 // 2), lambda i: (0, i, 0)),
            pl.BlockSpec((NC, B, H1 // 2), lambda i: (0, i, 0)),
            pl.BlockSpec((B, 1), lambda i: (i, 0)),
            pl.BlockSpec((B, 1), lambda i: (i, 0)),
            pl.BlockSpec((1, H1), lambda i: (0, 0)),
            pl.BlockSpec((H1, H2), lambda i: (0, 0)),
            pl.BlockSpec((8, 128), lambda i: (0, 0)),
        ],
        out_specs=pl.BlockSpec((B, H2), lambda i: (i, 0)),
        out_shape=jax.ShapeDtypeStruct((NPAD, H2), jnp.float32),
    )(aggp_a, aggp_b, ii_col, io_col, b1r, w2, dep)


def _h2_tc(aggp, ii_col, b2r):
    B = 1000

    def body(a_ref, ii_ref, b_ref, o_ref):
        a = a_ref[0] + a_ref[1]
        o_ref[...] = jnp.maximum(a * ii_ref[...] + b_ref[...], 0.0)

    return pl.pallas_call(
        body,
        grid=(N // B,),
        in_specs=[
            pl.BlockSpec((NC, B, H2), lambda i: (0, i, 0)),
            pl.BlockSpec((B, 1), lambda i: (i, 0)),
            pl.BlockSpec((1, H2), lambda i: (0, 0)),
        ],
        out_specs=pl.BlockSpec((B, H2), lambda i: (i, 0)),
        out_shape=jax.ShapeDtypeStruct((N, H2), jnp.float32),
    )(aggp, ii_col, b2r)


def _readout_tc(h2, w3b, b3r):
    B = 1000
    W = H2 * C

    def body(h_ref, w_ref, b_ref, o_ref):
        i = pl.program_id(0)
        m = lax.dot_general(h_ref[...], w_ref[...], (((0,), (0,)), ((), ())),
                            preferred_element_type=jnp.float32)
        k_iota = lax.broadcasted_iota(jnp.int32, (H2, W), 1)
        f_iota = lax.broadcasted_iota(jnp.int32, (H2, W), 0)
        stripe = k_iota - C * f_iota
        parts = [
            jnp.sum(jnp.where(stripe == c, m, 0.0), dtype=jnp.float32)
            for c in range(C)
        ]
        part = jnp.stack(parts).reshape(1, C)

        @pl.when(i == 0)
        def _():
            o_ref[...] = b_ref[...] + part

        @pl.when(i > 0)
        def _():
            o_ref[...] += part

    return pl.pallas_call(
        body,
        grid=(N // B,),
        in_specs=[
            pl.BlockSpec((B, H2), lambda i: (i, 0)),
            pl.BlockSpec((B, W), lambda i: (i, 0)),
            pl.BlockSpec((1, C), lambda i: (0, 0)),
        ],
        out_specs=pl.BlockSpec((1, C), lambda i: (0, 0)),
        out_shape=jax.ShapeDtypeStruct((1, C), jnp.float32),
    )(h2, w3b, b3r)


def kernel(inputs, edge_index, W1, b1, W2, b2, W3, b3):
    pad = NW * EPW - E
    src = jnp.concatenate(
        [edge_index[0], jnp.full((pad,), N, jnp.int32)]).reshape(NW, NB, EPB)
    dst = jnp.concatenate(
        [edge_index[1], jnp.full((pad,), N, jnp.int32)]).reshape(NW, NB, EPB)
    edges = jnp.stack([src, dst])

    degp = _deg_sc(edges)
    invs = _invs_tc(degp)
    io_col = invs[0][:, None]
    ii_col = invs[1][:, None]

    xp = jnp.concatenate(
        [inputs, jnp.zeros((NPAD - N, F_IN), jnp.float32)], axis=0)
    y1a, y1b = _xw_scale_tc(xp, W1, io_col)
    aggp1a, aggp1b = _agg2x_sc(y1a, y1b, edges)
    w3b = W3.reshape(N, H2 * C)
    y2 = _layer2_tc(aggp1a, aggp1b, ii_col, io_col, b1[None], W2, w3b)
    aggp2 = _agg_sc(y2, edges, H2, staged=True)
    h2 = _h2_tc(aggp2, ii_col, b2[None])
    out = _readout_tc(h2, w3b, b3[None])
    return out.reshape(C)
